# Initial kernel scaffold; baseline (speedup 1.0000x reference)
import jax
import jax.numpy as jnp
from jax.experimental import pallas as pl


def kernel(pos_u, pos_v, neg_v, u_weight, v_weight):
    raise NotImplementedError("write your pallas kernel here")


# speedup vs baseline: 2.6557x; 2.6557x over previous
import functools

import jax
import jax.numpy as jnp
from jax import lax
from jax.experimental import pallas as pl
from jax.experimental.pallas import tpu as pltpu
from jax.experimental.pallas import tpu_sc as plsc

PAIR = 28
D = 128
B = 4096
NEG = 20
EMB = 100000

NC = 2
NS = 16
NW = NC * NS
BPW = B // NW
NEG_PW = BPW * NEG
GROUP = 128
NGRP = NEG_PW // GROUP

_mesh = plsc.VectorSubcoreMesh(core_axis_name="c", subcore_axis_name="s")


@functools.partial(
    pl.kernel,
    mesh=_mesh,
    compiler_params=pltpu.CompilerParams(needs_layout_passes=False),
    out_type=[
        jax.ShapeDtypeStruct((NW, BPW, 16), jnp.float32),
        jax.ShapeDtypeStruct((NW, NGRP, GROUP, 16), jnp.float32),
    ],
    scratch_types=[
        pltpu.VMEM((BPW,), jnp.int32),
        pltpu.VMEM((BPW,), jnp.int32),
        pltpu.VMEM((NGRP, GROUP), jnp.int32),
        pltpu.VMEM((BPW, D), jnp.float32),
        pltpu.VMEM((GROUP, D), jnp.float32),
        pltpu.VMEM((GROUP, 16), jnp.float32),
        pltpu.SemaphoreType.DMA,
    ],
)
def _sc_dots(posu_hbm, posv_hbm, negv_hbm, u_hbm, v_hbm,
             pos_out, neg_out,
             posu_v, posv_v, negv_v, urows_v, rows_v, part_v, sem):
    wid = lax.axis_index("s") * NC + lax.axis_index("c")

    pltpu.sync_copy(posu_hbm.at[wid], posu_v)
    pltpu.sync_copy(posv_hbm.at[wid], posv_v)
    pltpu.sync_copy(negv_hbm.at[wid], negv_v)
    pltpu.async_copy(u_hbm.at[posu_v], urows_v, sem).wait()

    lanes = lax.iota(jnp.int32, 16)
    cols = [lanes + 16 * j for j in range(8)]

    def row_dot(u_row_vec, row_i_vec):
        acc = None
        for j in range(8):
            u_j = plsc.load_gather(urows_v, [u_row_vec, cols[j]])
            r_j = plsc.load_gather(rows_v, [row_i_vec, cols[j]])
            acc = u_j * r_j if acc is None else acc + u_j * r_j
        return acc

    pltpu.async_copy(v_hbm.at[posv_v], rows_v, sem).wait()

    def pos_body(i, _):
        iv = jnp.full((16,), i, dtype=jnp.int32)
        acc = row_dot(iv, iv)
        plsc.store_scatter(part_v, [iv, lanes], acc)
        return _

    lax.fori_loop(0, GROUP, pos_body, 0)
    pltpu.sync_copy(part_v, pos_out.at[wid])

    def neg_group(g, _):
        pltpu.async_copy(v_hbm.at[negv_v.at[g]], rows_v, sem).wait()

        def neg_body(i, _c):
            iv = jnp.full((16,), i, dtype=jnp.int32)
            bv = jnp.full((16,), lax.div(g * GROUP + i, NEG), dtype=jnp.int32)
            acc = row_dot(bv, iv)
            plsc.store_scatter(part_v, [iv, lanes], acc)
            return _c

        lax.fori_loop(0, GROUP, neg_body, 0)
        pltpu.sync_copy(part_v, neg_out.at[wid, g])
        return _

    lax.fori_loop(0, NGRP, neg_group, 0)


def _tc_reduce_body(neg_ref, pos_ref, out_ref):
    r = lax.broadcasted_iota(jnp.int32, (D, 8), 0) // 16
    c = lax.broadcasted_iota(jnp.int32, (D, 8), 1)
    m = (r == c).astype(jnp.float32)

    def lsig(x):
        return jnp.minimum(x, 0.0) - jnp.log1p(jnp.exp(-jnp.abs(x)))

    neg_s = jnp.dot(neg_ref[...], m, preferred_element_type=jnp.float32)
    pos_s = jnp.dot(pos_ref[...], m, preferred_element_type=jnp.float32)
    out_ref[0, 0] = -(jnp.sum(lsig(pos_s)) + jnp.sum(lsig(-neg_s)))


_tc_reduce = pl.pallas_call(
    _tc_reduce_body,
    out_shape=jax.ShapeDtypeStruct((1, 1), jnp.float32),
    out_specs=pl.BlockSpec(memory_space=pltpu.SMEM),
)


def kernel(pos_u, pos_v, neg_v, u_weight, v_weight):
    pos_u = pos_u.astype(jnp.int32)
    pos_v = pos_v.astype(jnp.int32)
    neg_v = neg_v.astype(jnp.int32)

    posu_w = pos_u.reshape(NW, BPW)
    posv_w = pos_v.reshape(NW, BPW)
    negv_w = neg_v.reshape(NW, NGRP, GROUP)

    pos_part, neg_part = _sc_dots(posu_w, posv_w, negv_w, u_weight, v_weight)

    neg2 = neg_part.reshape(NW * NGRP * GROUP * 16 // D, D)
    pos2 = pos_part.reshape(NW * BPW * 16 // D, D)
    loss = _tc_reduce(neg2, pos2)
    return loss[0, 0]
